# baseline (speedup 1.0000x reference)
import jax
import jax.numpy as jnp
from jax import lax
from jax.experimental import pallas as pl
from jax.experimental.pallas import tpu as pltpu
from jax.experimental.pallas import tpu_sc as plsc

_POOL = 225
_K = 32
_HALF = 16
_NEG = -1e30


def _score_select_kernel(infeat_ref, idx_ref, val_ref):
    x0 = infeat_ref[0, 0]
    x1 = infeat_ref[0, 1]
    m = jnp.maximum(x0, x1)
    e0 = jnp.exp(x0 - m)
    e1 = jnp.exp(x1 - m)
    inv_s = 1.0 / (e0 + e1)
    p0 = e0 * inv_s
    p1 = e1 * inv_s
    ent = -(p0 * jnp.log(p0 + 1e-5) + p1 * jnp.log(p1 + 1e-5))

    rows = jax.lax.broadcasted_iota(jnp.int32, (256, 256), 0)
    cols = jax.lax.broadcasted_iota(jnp.int32, (256, 256), 1)
    band = ((rows >= cols) & (rows < cols + _K) & (cols < _POOL)).astype(
        jnp.float32)
    valid = (rows < _POOL) & (cols < _POOL)
    flat = rows * _POOL + cols

    idxs = []
    vals = []
    for c in range(2):
        g = (p0 if c == 0 else p1) - 0.1 * ent
        tmp = jax.lax.dot_general(
            g, band, (((1,), (0,)), ((), ())),
            preferred_element_type=jnp.float32,
            precision=jax.lax.Precision.HIGHEST)
        score = jax.lax.dot_general(
            band, tmp, (((0,), (0,)), ((), ())),
            preferred_element_type=jnp.float32,
            precision=jax.lax.Precision.HIGHEST)
        score = jnp.where(valid, score * (1.0 / (_K * _K)), _NEG)
        for kk in range(2):
            v = jnp.max(score)
            idx = jnp.min(jnp.where(score == v, flat, jnp.int32(2**31 - 1)))
            py = idx // _POOL
            px = idx - py * _POOL
            idxs.append(idx)
            vals.append(v)
            in_box = ((rows >= py - _HALF) & (rows < py + _HALF) &
                      (cols >= px - _HALF) & (cols < px + _HALF) & valid)
            score = jnp.where(in_box, 0.0, score)
    idx_ref[...] = jnp.stack(idxs).reshape(1, 1, 4)
    val_ref[...] = jnp.stack(vals).reshape(1, 1, 4)


_SC_MESH = plsc.VectorSubcoreMesh(core_axis_name="c", subcore_axis_name="s")


def _sc_gather(pref_hbm, tab_f, tab_i, tab_lt, tab_lps,
               cls_out, fda_out, lt_out, lps_out,
               pref_v, idx_ref, slab, out_v, sem):
    w = lax.axis_index("s") * 2 + lax.axis_index("c")
    pltpu.sync_copy(pref_hbm.at[w], pref_v)
    iota = lax.iota(jnp.int32, 16)
    d = pref_v[pl.ds(0, 16)]
    base_f = pref_v[pl.ds(16, 16)]
    base_i = pref_v[pl.ds(32, 16)]
    base_l = pref_v[pl.ds(48, 16)]
    t2 = jnp.where(d == 0, 0, jnp.int32(2))

    pvec = []
    pmask = []
    for t in range(3):
        xcol = 16 * t - d + iota
        pvec.append(xcol)
        pmask.append((xcol >= 0) & (xcol < 32))

    def build_idx(j, base_c):
        for t in range(3):
            tt = jnp.int32(t) if t < 2 else t2
            for hh in range(2):
                idx_ref[j, pl.ds(t * 32 + 16 * hh, 16)] = (
                    base_c + iota * 16 + 256 * hh + tt)

    def extract(g, wbase):
        for dy in range(32):
            for t in range(3):
                v = slab[g * 96 + t * 32 + dy]
                plsc.store_scatter(out_v, [pvec[t] + (wbase + dy * 32)],
                                   v, mask=pmask[t])

    def fda_group(G, carry):
        for j in range(8):
            build_idx(j, base_f + (G * 8 + j) * 4096)
        cps = [pltpu.async_copy(tab_f.at[idx_ref.at[j]],
                                slab.at[pl.ds(j * 96, 96)], sem)
               for j in range(8)]
        for c in cps:
            c.wait()
        for g in range(8):
            extract(g, g * 1024)
        pltpu.sync_copy(out_v, fda_out.at[pl.ds(w * 131072 + G * 8192, 8192)])
        return carry

    lax.fori_loop(0, 16, fda_group, 0)

    build_idx(0, base_i)
    build_idx(1, base_i + 4096)
    build_idx(2, base_l)
    build_idx(3, base_l)
    tabs = (tab_i, tab_i, tab_lt, tab_lps)
    cps = [pltpu.async_copy(tabs[j].at[idx_ref.at[j]],
                            slab.at[pl.ds(j * 96, 96)], sem)
           for j in range(4)]
    for c in cps:
        c.wait()
    for g in range(4):
        extract(g, g * 1024)
    pltpu.sync_copy(out_v.at[pl.ds(0, 2048)], cls_out.at[pl.ds(w * 2048, 2048)])
    pltpu.sync_copy(out_v.at[pl.ds(2048, 1024)],
                    lt_out.at[pl.ds(w * 1024, 1024)])
    pltpu.sync_copy(out_v.at[pl.ds(3072, 1024)],
                    lps_out.at[pl.ds(w * 1024, 1024)])


def _run_sc_gather(pref16, infeat, labelTpesudo, labelT, FeatureDA):
    f = pl.kernel(
        _sc_gather,
        out_type=[
            jax.ShapeDtypeStruct((32 * 2 * _K * _K,), jnp.float32),
            jax.ShapeDtypeStruct((32 * 128 * _K * _K,), jnp.float32),
            jax.ShapeDtypeStruct((32 * 1 * _K * _K,), jnp.float32),
            jax.ShapeDtypeStruct((32 * 1 * _K * _K,), jnp.float32),
        ],
        mesh=_SC_MESH,
        scratch_types=[
            pltpu.MemorySpace.VMEM((64,), jnp.int32),
            pltpu.MemorySpace.VMEM((8, 96), jnp.int32),
            pltpu.MemorySpace.VMEM((768, 16), jnp.float32),
            pltpu.MemorySpace.VMEM((8192,), jnp.float32),
            pltpu.SemaphoreType.DMA,
        ],
        compiler_params=pltpu.CompilerParams(
            needs_layout_passes=False, use_tc_tiling_on_sc=False),
    )
    cls, fda, lt, lps = f(
        pref16,
        FeatureDA.reshape(8 * 128 * 256 * 16, 16),
        infeat.reshape(8 * 2 * 256 * 16, 16),
        labelT.reshape(8 * 256 * 16, 16),
        labelTpesudo.reshape(8 * 256 * 16, 16),
    )
    return (cls.reshape(32, 2, _K, _K), fda.reshape(32, 128, _K, _K),
            lt.reshape(32, 1, _K, _K), lps.reshape(32, 1, _K, _K))


def kernel(infeat, labelTpesudo, labelT, FeatureDA):
    idx8, vals8 = pl.pallas_call(
        _score_select_kernel,
        grid=(8,),
        in_specs=[pl.BlockSpec((1, 2, 256, 256), lambda b: (b, 0, 0, 0))],
        out_specs=[pl.BlockSpec((1, 1, 4), lambda b: (b, 0, 0)),
                   pl.BlockSpec((1, 1, 4), lambda b: (b, 0, 0))],
        out_shape=[jax.ShapeDtypeStruct((8, 1, 4), jnp.int32),
                   jax.ShapeDtypeStruct((8, 1, 4), jnp.float32)],
    )(infeat)

    idx = idx8.reshape(8, 2, 2)
    vals = vals8.reshape(8, 2, 2)
    idxn = jnp.transpose(idx, (1, 2, 0)).reshape(32)
    provalue = jnp.transpose(vals, (1, 2, 0)).reshape(32)
    py = idxn // _POOL
    px = idxn - py * _POOL
    bn = jnp.arange(32, dtype=jnp.int32) % 8
    col16 = px // 16
    d = px - col16 * 16
    base_f = ((bn * 128) * 256 + py) * 16 + col16
    base_i = ((bn * 2) * 256 + py) * 16 + col16
    base_l = (bn * 256 + py) * 16 + col16
    pref16 = jnp.broadcast_to(
        jnp.stack([d, base_f, base_i, base_l], axis=1).astype(jnp.int32)
        [:, :, None], (32, 4, 16)).reshape(32, 64)

    cls, fda, lt, lps = _run_sc_gather(
        pref16, infeat, labelTpesudo, labelT, FeatureDA)
    return (cls, fda, lt, lps, provalue)


# speedup vs baseline: 1.0618x; 1.0618x over previous
import jax
import jax.numpy as jnp
from jax import lax
from jax.experimental import pallas as pl
from jax.experimental.pallas import tpu as pltpu
from jax.experimental.pallas import tpu_sc as plsc

_POOL = 225
_K = 32
_HALF = 16
_NEG = -1e30


def _score_select_kernel(infeat_ref, idx_ref, val_ref):
    x0 = infeat_ref[0, 0]
    x1 = infeat_ref[0, 1]
    m = jnp.maximum(x0, x1)
    e0 = jnp.exp(x0 - m)
    e1 = jnp.exp(x1 - m)
    inv_s = 1.0 / (e0 + e1)
    p0 = e0 * inv_s
    p1 = e1 * inv_s
    ent = -(p0 * jnp.log(p0 + 1e-5) + p1 * jnp.log(p1 + 1e-5))

    rows = jax.lax.broadcasted_iota(jnp.int32, (256, 256), 0)
    cols = jax.lax.broadcasted_iota(jnp.int32, (256, 256), 1)
    band = ((rows >= cols) & (rows < cols + _K) & (cols < _POOL)).astype(
        jnp.float32)
    valid = (rows < _POOL) & (cols < _POOL)
    flat = rows * _POOL + cols

    idxs = []
    vals = []
    for c in range(2):
        g = (p0 if c == 0 else p1) - 0.1 * ent
        tmp = jax.lax.dot_general(
            g, band, (((1,), (0,)), ((), ())),
            preferred_element_type=jnp.float32,
            precision=jax.lax.Precision.HIGHEST)
        score = jax.lax.dot_general(
            band, tmp, (((0,), (0,)), ((), ())),
            preferred_element_type=jnp.float32,
            precision=jax.lax.Precision.HIGHEST)
        score = jnp.where(valid, score * (1.0 / (_K * _K)), _NEG)
        for kk in range(2):
            v = jnp.max(score)
            idx = jnp.min(jnp.where(score == v, flat, jnp.int32(2**31 - 1)))
            py = idx // _POOL
            px = idx - py * _POOL
            idxs.append(idx)
            vals.append(v)
            in_box = ((rows >= py - _HALF) & (rows < py + _HALF) &
                      (cols >= px - _HALF) & (cols < px + _HALF) & valid)
            score = jnp.where(in_box, 0.0, score)
    idx_ref[...] = jnp.stack(idxs).reshape(1, 1, 4)
    val_ref[...] = jnp.stack(vals).reshape(1, 1, 4)


_SC_MESH = plsc.VectorSubcoreMesh(core_axis_name="c", subcore_axis_name="s")


def _sc_gather(pref_hbm, tab_f, tab_i, tab_lt, tab_lps,
               cls_out, fda_out, lt_out, lps_out,
               pref_v, idx_a, idx_b, idx_s, slab_a, slab_b,
               out_a, out_b, sem_a, sem_b):
    w = lax.axis_index("s") * 2 + lax.axis_index("c")
    pltpu.sync_copy(pref_hbm.at[w], pref_v)
    iota = lax.iota(jnp.int32, 16)
    d = pref_v[pl.ds(0, 16)]
    base_f = pref_v[pl.ds(16, 16)]
    base_i = pref_v[pl.ds(32, 16)]
    base_l = pref_v[pl.ds(48, 16)]
    t2 = jnp.where(d == 0, 0, jnp.int32(2))

    pvec = []
    pmask = []
    for t in range(3):
        xcol = 16 * t - d + iota
        pvec.append(xcol)
        pmask.append((xcol >= 0) & (xcol < 32))

    def build_idx(idx_r, G):
        for j in range(8):
            base_c = base_f + (G * 8 + j) * 4096
            for t in range(3):
                tt = jnp.int32(t) if t < 2 else t2
                for hh in range(2):
                    idx_r[pl.ds(j * 96 + t * 32 + 16 * hh, 16)] = (
                        base_c + iota * 16 + 256 * hh + tt)

    def extract(slab_r, out_r):
        for g in range(8):
            for dy in range(32):
                for t in range(3):
                    v = slab_r[g * 96 + t * 32 + dy]
                    plsc.store_scatter(
                        out_r, [pvec[t] + (g * 1024 + dy * 32)],
                        v, mask=pmask[t])

    def fire(idx_r, slab_r, sem_r):
        return pltpu.make_async_copy(tab_f.at[idx_r], slab_r, sem_r)

    build_idx(idx_a, 0)
    fire(idx_a, slab_a, sem_a).start()

    def pipe(P, carry):
        g0 = 2 * P
        build_idx(idx_b, g0 + 1)
        fire(idx_b, slab_b, sem_b).start()
        fire(idx_a, slab_a, sem_a).wait()
        extract(slab_a, out_a)

        @pl.when(P < 7)
        def _():
            build_idx(idx_a, g0 + 2)
            fire(idx_a, slab_a, sem_a).start()

        pltpu.sync_copy(out_a, fda_out.at[pl.ds(w * 131072 + g0 * 8192, 8192)])
        fire(idx_b, slab_b, sem_b).wait()
        extract(slab_b, out_b)
        pltpu.sync_copy(
            out_b, fda_out.at[pl.ds(w * 131072 + (g0 + 1) * 8192, 8192)])
        return carry

    lax.fori_loop(0, 8, pipe, 0)

    def build_small(j, base_c):
        for t in range(3):
            tt = jnp.int32(t) if t < 2 else t2
            for hh in range(2):
                idx_s[j, pl.ds(t * 32 + 16 * hh, 16)] = (
                    base_c + iota * 16 + 256 * hh + tt)

    build_small(0, base_i)
    build_small(1, base_i + 4096)
    build_small(2, base_l)
    build_small(3, base_l)
    tabs = (tab_i, tab_i, tab_lt, tab_lps)
    cps = [pltpu.async_copy(tabs[j].at[idx_s.at[j]],
                            slab_a.at[pl.ds(j * 96, 96)], sem_a)
           for j in range(4)]
    for c in cps:
        c.wait()
    for g in range(4):
        for dy in range(32):
            for t in range(3):
                v = slab_a[g * 96 + t * 32 + dy]
                plsc.store_scatter(out_a, [pvec[t] + (g * 1024 + dy * 32)],
                                   v, mask=pmask[t])
    pltpu.sync_copy(out_a.at[pl.ds(0, 2048)], cls_out.at[pl.ds(w * 2048, 2048)])
    pltpu.sync_copy(out_a.at[pl.ds(2048, 1024)],
                    lt_out.at[pl.ds(w * 1024, 1024)])
    pltpu.sync_copy(out_a.at[pl.ds(3072, 1024)],
                    lps_out.at[pl.ds(w * 1024, 1024)])


def _run_sc_gather(pref16, infeat, labelTpesudo, labelT, FeatureDA):
    f = pl.kernel(
        _sc_gather,
        out_type=[
            jax.ShapeDtypeStruct((32 * 2 * _K * _K,), jnp.float32),
            jax.ShapeDtypeStruct((32 * 128 * _K * _K,), jnp.float32),
            jax.ShapeDtypeStruct((32 * 1 * _K * _K,), jnp.float32),
            jax.ShapeDtypeStruct((32 * 1 * _K * _K,), jnp.float32),
        ],
        mesh=_SC_MESH,
        scratch_types=[
            pltpu.MemorySpace.VMEM((64,), jnp.int32),
            pltpu.MemorySpace.VMEM((768,), jnp.int32),
            pltpu.MemorySpace.VMEM((768,), jnp.int32),
            pltpu.MemorySpace.VMEM((4, 96), jnp.int32),
            pltpu.MemorySpace.VMEM((768, 16), jnp.float32),
            pltpu.MemorySpace.VMEM((768, 16), jnp.float32),
            pltpu.MemorySpace.VMEM((8192,), jnp.float32),
            pltpu.MemorySpace.VMEM((8192,), jnp.float32),
            pltpu.SemaphoreType.DMA,
            pltpu.SemaphoreType.DMA,
        ],
        compiler_params=pltpu.CompilerParams(
            needs_layout_passes=False, use_tc_tiling_on_sc=False),
    )
    cls, fda, lt, lps = f(
        pref16,
        FeatureDA.reshape(8 * 128 * 256 * 16, 16),
        infeat.reshape(8 * 2 * 256 * 16, 16),
        labelT.reshape(8 * 256 * 16, 16),
        labelTpesudo.reshape(8 * 256 * 16, 16),
    )
    return (cls.reshape(32, 2, _K, _K), fda.reshape(32, 128, _K, _K),
            lt.reshape(32, 1, _K, _K), lps.reshape(32, 1, _K, _K))


def kernel(infeat, labelTpesudo, labelT, FeatureDA):
    idx8, vals8 = pl.pallas_call(
        _score_select_kernel,
        grid=(8,),
        in_specs=[pl.BlockSpec((1, 2, 256, 256), lambda b: (b, 0, 0, 0))],
        out_specs=[pl.BlockSpec((1, 1, 4), lambda b: (b, 0, 0)),
                   pl.BlockSpec((1, 1, 4), lambda b: (b, 0, 0))],
        out_shape=[jax.ShapeDtypeStruct((8, 1, 4), jnp.int32),
                   jax.ShapeDtypeStruct((8, 1, 4), jnp.float32)],
    )(infeat)

    idx = idx8.reshape(8, 2, 2)
    vals = vals8.reshape(8, 2, 2)
    idxn = jnp.transpose(idx, (1, 2, 0)).reshape(32)
    provalue = jnp.transpose(vals, (1, 2, 0)).reshape(32)
    py = idxn // _POOL
    px = idxn - py * _POOL
    bn = jnp.arange(32, dtype=jnp.int32) % 8
    col16 = px // 16
    d = px - col16 * 16
    base_f = ((bn * 128) * 256 + py) * 16 + col16
    base_i = ((bn * 2) * 256 + py) * 16 + col16
    base_l = (bn * 256 + py) * 16 + col16
    pref16 = jnp.broadcast_to(
        jnp.stack([d, base_f, base_i, base_l], axis=1).astype(jnp.int32)
        [:, :, None], (32, 4, 16)).reshape(32, 64)

    cls, fda, lt, lps = _run_sc_gather(
        pref16, infeat, labelTpesudo, labelT, FeatureDA)
    return (cls, fda, lt, lps, provalue)


# speedup vs baseline: 1.5602x; 1.4693x over previous
import jax
import jax.numpy as jnp
from jax.experimental import pallas as pl
from jax.experimental.pallas import tpu as pltpu

_POOL = 225
_K = 32
_HALF = 16
_NEG = -1e30


def _score_select_kernel(infeat_ref, idx_ref, val_ref):
    x0 = infeat_ref[0, 0]
    x1 = infeat_ref[0, 1]
    m = jnp.maximum(x0, x1)
    e0 = jnp.exp(x0 - m)
    e1 = jnp.exp(x1 - m)
    inv_s = 1.0 / (e0 + e1)
    p0 = e0 * inv_s
    p1 = e1 * inv_s
    ent = -(p0 * jnp.log(p0 + 1e-5) + p1 * jnp.log(p1 + 1e-5))

    rows = jax.lax.broadcasted_iota(jnp.int32, (256, 256), 0)
    cols = jax.lax.broadcasted_iota(jnp.int32, (256, 256), 1)
    band = ((rows >= cols) & (rows < cols + _K) & (cols < _POOL)).astype(
        jnp.float32)
    valid = (rows < _POOL) & (cols < _POOL)
    flat = rows * _POOL + cols

    idxs = []
    vals = []
    for c in range(2):
        g = (p0 if c == 0 else p1) - 0.1 * ent
        tmp = jax.lax.dot_general(
            g, band, (((1,), (0,)), ((), ())),
            preferred_element_type=jnp.float32,
            precision=jax.lax.Precision.HIGHEST)
        score = jax.lax.dot_general(
            band, tmp, (((0,), (0,)), ((), ())),
            preferred_element_type=jnp.float32,
            precision=jax.lax.Precision.HIGHEST)
        score = jnp.where(valid, score * (1.0 / (_K * _K)), _NEG)
        for kk in range(2):
            v = jnp.max(score)
            idx = jnp.min(jnp.where(score == v, flat, jnp.int32(2**31 - 1)))
            py = idx // _POOL
            px = idx - py * _POOL
            idxs.append(idx)
            vals.append(v)
            in_box = ((rows >= py - _HALF) & (rows < py + _HALF) &
                      (cols >= px - _HALF) & (cols < px + _HALF) & valid)
            score = jnp.where(in_box, 0.0, score)
    idx_ref[...] = jnp.stack(idxs).reshape(1, 1, 4)
    val_ref[...] = jnp.stack(vals).reshape(1, 1, 4)


def _extract(buf, slot, dy, px, sel):
    xr = pltpu.roll(buf[slot], jax.lax.rem(40 - dy, 40), axis=1)[:, :_K, :]
    return jax.lax.dot_general(
        xr, sel, (((2,), (0,)), ((), ())),
        preferred_element_type=jnp.float32,
        precision=jax.lax.Precision.HIGHEST)


def _gather_kernel(pref_ref, infeat_hbm, lps_hbm, lt_hbm, fda_hbm,
                   cls_out, fda_out, lt_out, lps_out,
                   s_if, s_fda, s_lt, s_lps,
                   sems):
    n = pl.program_id(0)
    bufs = (s_if, s_fda, s_lt, s_lps)
    srcs = (infeat_hbm, fda_hbm, lt_hbm, lps_hbm)

    def copies(m, slot):
        b = pref_ref[m, 0]
        py0 = (pref_ref[m, 1] // 8) * 8
        return [
            pltpu.make_async_copy(
                src.at[b, :, pl.ds(py0, 40), :], buf.at[slot], sems.at[slot, j])
            for j, (src, buf) in enumerate(zip(srcs, bufs))
        ]

    @pl.when(n == 0)
    def _():
        for c in copies(0, 0):
            c.start()

    @pl.when(n + 1 < 32)
    def _():
        for c in copies(n + 1, (n + 1) % 2):
            c.start()

    slot = n % 2
    px = pref_ref[n, 2]
    dy = pref_ref[n, 1] - (pref_ref[n, 1] // 8) * 8
    pz = jax.lax.broadcasted_iota(jnp.int32, (256, _K), 0)
    jz = jax.lax.broadcasted_iota(jnp.int32, (256, _K), 1)
    sel = (pz == jz + px).astype(jnp.float32)
    for c, (buf, out) in zip(copies(n, slot),
                             ((s_if, cls_out), (s_fda, fda_out),
                              (s_lt, lt_out), (s_lps, lps_out))):
        c.wait()
        out[0] = _extract(buf, slot, dy, px, sel)


def kernel(infeat, labelTpesudo, labelT, FeatureDA):
    idx8, vals8 = pl.pallas_call(
        _score_select_kernel,
        grid=(8,),
        in_specs=[pl.BlockSpec((1, 2, 256, 256), lambda b: (b, 0, 0, 0))],
        out_specs=[pl.BlockSpec((1, 1, 4), lambda b: (b, 0, 0)),
                   pl.BlockSpec((1, 1, 4), lambda b: (b, 0, 0))],
        out_shape=[jax.ShapeDtypeStruct((8, 1, 4), jnp.int32),
                   jax.ShapeDtypeStruct((8, 1, 4), jnp.float32)],
    )(infeat)

    idx = idx8.reshape(8, 2, 2)
    vals = vals8.reshape(8, 2, 2)
    idxn = jnp.transpose(idx, (1, 2, 0)).reshape(32)
    provalue = jnp.transpose(vals, (1, 2, 0)).reshape(32)
    py = idxn // _POOL
    px = idxn - py * _POOL
    bn = jnp.arange(32, dtype=jnp.int32) % 8
    pref = jnp.stack([bn, py, px], axis=1).astype(jnp.int32)

    grid_spec = pltpu.PrefetchScalarGridSpec(
        num_scalar_prefetch=1,
        grid=(32,),
        in_specs=[pl.BlockSpec(memory_space=pl.ANY)] * 4,
        out_specs=[
            pl.BlockSpec((1, 2, _K, _K), lambda n, pref: (n, 0, 0, 0)),
            pl.BlockSpec((1, 128, _K, _K), lambda n, pref: (n, 0, 0, 0)),
            pl.BlockSpec((1, 1, _K, _K), lambda n, pref: (n, 0, 0, 0)),
            pl.BlockSpec((1, 1, _K, _K), lambda n, pref: (n, 0, 0, 0)),
        ],
        scratch_shapes=[
            pltpu.MemorySpace.VMEM((2, 2, 40, 256), jnp.float32),
            pltpu.MemorySpace.VMEM((2, 128, 40, 256), jnp.float32),
            pltpu.MemorySpace.VMEM((2, 1, 40, 256), jnp.float32),
            pltpu.MemorySpace.VMEM((2, 1, 40, 256), jnp.float32),
            pltpu.SemaphoreType.DMA((2, 4)),
        ],
    )
    cls, fda, lt, lps = pl.pallas_call(
        _gather_kernel,
        grid_spec=grid_spec,
        out_shape=[
            jax.ShapeDtypeStruct((32, 2, _K, _K), jnp.float32),
            jax.ShapeDtypeStruct((32, 128, _K, _K), jnp.float32),
            jax.ShapeDtypeStruct((32, 1, _K, _K), jnp.float32),
            jax.ShapeDtypeStruct((32, 1, _K, _K), jnp.float32),
        ],
    )(pref, infeat, labelTpesudo, labelT, FeatureDA)

    return (cls, fda, lt, lps, provalue)
